# baseline (speedup 1.0000x reference)
import functools
import numpy as np
import jax
import jax.numpy as jnp
from jax.experimental import pallas as pl
from jax.experimental.pallas import tpu as pltpu

_VMEM_LIMIT = 64 * 1024 * 1024


def _rup(x, m):
    return (x + m - 1) // m * m


def _mm_kernel(a_ref, b_ref, sc_ref, sh_ref, res_ref, o_ref, acc_ref, *,
               relu, affine, residual, nk):
    if nk > 1:
        @pl.when(pl.program_id(2) == 0)
        def _():
            acc_ref[...] = jnp.zeros_like(acc_ref)
        acc_ref[...] += jnp.dot(a_ref[...], b_ref[...],
                                preferred_element_type=jnp.float32)

        @pl.when(pl.program_id(2) == nk - 1)
        def _():
            acc = acc_ref[...]
            if affine:
                acc = acc * sc_ref[...] + sh_ref[...]
            if residual:
                acc = acc + res_ref[...].astype(jnp.float32)
            if relu:
                acc = jnp.maximum(acc, 0.0)
            o_ref[...] = acc.astype(o_ref.dtype)
    else:
        acc = jnp.dot(a_ref[...], b_ref[...],
                      preferred_element_type=jnp.float32)
        if affine:
            acc = acc * sc_ref[...] + sh_ref[...]
        if residual:
            acc = acc + res_ref[...].astype(jnp.float32)
        if relu:
            acc = jnp.maximum(acc, 0.0)
        o_ref[...] = acc.astype(o_ref.dtype)


def _matmul(a, b, *, scale=None, shift=None, residual=None, relu=False,
            out_dtype=jnp.bfloat16):
    M, K = a.shape
    Kb, N = b.shape
    assert K == Kb, (a.shape, b.shape)
    Np = _rup(N, 128)
    Mp = _rup(M, 8)

    if Mp >= 512:
        tm = 256
    elif Mp >= 128:
        tm = Mp // 2 if Mp % 2 == 0 and (Mp // 2) % 8 == 0 else Mp
    else:
        tm = Mp
    Mp = _rup(Mp, tm)
    if Np >= 1024:
        tn = 512
    elif Np >= 256:
        tn = 256 if (Mp // tm) * (Np // 256) >= 2 else 128
    else:
        tn = 128
    while Np % tn:
        tn //= 2
    Kp = _rup(K, 8)
    if Kp > 1024:
        tk = next((t for t in (1024, 512, 256) if Kp % t == 0), Kp)
    else:
        tk = Kp
    nk = Kp // tk

    a = a.astype(jnp.bfloat16)
    b = b.astype(jnp.bfloat16)
    if Kp > K:
        a = jnp.pad(a, ((0, 0), (0, Kp - K)))
        b = jnp.pad(b, ((0, Kp - K), (0, 0)))
    if Mp > M:
        a = jnp.pad(a, ((0, Mp - M), (0, 0)))
    if Np > N:
        b = jnp.pad(b, ((0, 0), (0, Np - N)))

    grid = (Mp // tm, Np // tn, nk)
    a_spec = pl.BlockSpec((tm, tk), lambda i, j, k: (i, k))
    b_spec = pl.BlockSpec((tk, tn), lambda i, j, k: (k, j))
    o_spec = pl.BlockSpec((tm, tn), lambda i, j, k: (i, j))
    v_spec = pl.BlockSpec((1, tn), lambda i, j, k: (0, j))

    affine = scale is not None
    has_res = residual is not None
    ins = [a, b]
    in_specs = [a_spec, b_spec]
    if affine:
        sc = jnp.pad(scale.reshape(1, N).astype(jnp.float32),
                     ((0, 0), (0, Np - N)))
        sh = jnp.pad(shift.reshape(1, N).astype(jnp.float32),
                     ((0, 0), (0, Np - N)))
        ins += [sc, sh]
        in_specs += [v_spec, v_spec]
    if has_res:
        res = residual.astype(jnp.bfloat16)
        if res.shape != (Mp, Np):
            res = jnp.pad(res, ((0, Mp - res.shape[0]),
                                (0, Np - res.shape[1])))
        ins.append(res)
        in_specs.append(o_spec)

    kern = functools.partial(_mm_kernel, relu=relu, affine=affine,
                             residual=has_res, nk=nk)
    if not affine:
        kern = lambda a_r, b_r, o_r, ac_r: functools.partial(
            _mm_kernel, relu=relu, affine=False, residual=False, nk=nk)(
                a_r, b_r, None, None, None, o_r, ac_r)
    elif not has_res:
        kern = lambda a_r, b_r, s_r, h_r, o_r, ac_r: functools.partial(
            _mm_kernel, relu=relu, affine=True, residual=False, nk=nk)(
                a_r, b_r, s_r, h_r, None, o_r, ac_r)

    out = pl.pallas_call(
        kern, grid=grid,
        in_specs=in_specs, out_specs=o_spec,
        out_shape=jax.ShapeDtypeStruct((Mp, Np), out_dtype),
        scratch_shapes=[pltpu.VMEM((tm, tn), jnp.float32)],
        compiler_params=pltpu.CompilerParams(
            dimension_semantics=("parallel", "parallel", "arbitrary"),
            vmem_limit_bytes=_VMEM_LIMIT))(*ins)
    if Mp > M:
        out = out[:M]
    return out


def _conv3x3_kernel(x_ref, w_ref, sc_ref, sh_ref, o_ref, *, TR, wd, cin):
    nb, _, _, _ = x_ref.shape
    r0 = pl.program_id(1) * TR
    acc = jnp.zeros((nb * TR * wd, o_ref.shape[-1]), jnp.float32)
    for dh in range(3):
        v = x_ref[:, pl.ds(r0 + dh, TR), :, :]
        for dw in range(3):
            a = v[:, :, dw:dw + wd, :].reshape(nb * TR * wd, cin)
            t = dh * 3 + dw
            acc += jnp.dot(a, w_ref[t * cin:(t + 1) * cin, :],
                           preferred_element_type=jnp.float32)
    acc = acc * sc_ref[...] + sh_ref[...]
    acc = jnp.maximum(acc, 0.0).astype(o_ref.dtype)
    o_ref[...] = acc.reshape(o_ref.shape)


def _conv3x3_s1(x, w, scale, shift):
    n, h, wd, cin = x.shape
    kh, kw, cin_w, cout = w.shape
    if cin_w != cin:
        w = jnp.pad(w, ((0, 0), (0, 0), (0, cin - cin_w), (0, 0)))
    Np = _rup(cout, 128)
    P = _rup(wd + 2, 8)
    xp = jnp.pad(x, ((0, 0), (1, 1), (1, P - wd - 1), (0, 0)))

    TR = min(h, 16)
    nb = max(1, min(n, 256 // (TR * wd)))
    while n % nb:
        nb -= 1
    nr = h // TR

    wm = w.reshape(9 * cin, cout).astype(jnp.bfloat16)
    if Np > cout:
        wm = jnp.pad(wm, ((0, 0), (0, Np - cout)))
    sc = jnp.pad(scale.reshape(1, cout).astype(jnp.float32),
                 ((0, 0), (0, Np - cout)))
    sh = jnp.pad(shift.reshape(1, cout).astype(jnp.float32),
                 ((0, 0), (0, Np - cout)))

    tn = Np if (n // nb) * nr >= 4 or Np == 128 else 128
    grid = (n // nb, nr, Np // tn)
    out = pl.pallas_call(
        functools.partial(_conv3x3_kernel, TR=TR, wd=wd, cin=cin),
        grid=grid,
        in_specs=[
            pl.BlockSpec((nb, h + 2, P, cin), lambda b, r, j: (b, 0, 0, 0)),
            pl.BlockSpec((9 * cin, tn), lambda b, r, j: (0, j)),
            pl.BlockSpec((1, tn), lambda b, r, j: (0, j)),
            pl.BlockSpec((1, tn), lambda b, r, j: (0, j)),
        ],
        out_specs=pl.BlockSpec((nb, TR, wd, tn), lambda b, r, j: (b, r, 0, j)),
        out_shape=jax.ShapeDtypeStruct((n, h, wd, Np), jnp.bfloat16),
        compiler_params=pltpu.CompilerParams(
            dimension_semantics=("parallel", "parallel", "parallel"),
            vmem_limit_bytes=_VMEM_LIMIT))(xp, wm, sc, sh)
    return out


def _layer_kernel(*refs, blocks, nb, h, wd, planes, cin):
    x_ref = refs[0]
    o_ref = refs[-2]
    zp_ref = refs[-1]
    M = nb * h * wd
    x = x_ref[...].reshape(M, cin)
    ri = 1
    for has_down in blocks:
        w1, sc1, sh1, w2, sc2, sh2, w3, sc3, sh3 = refs[ri:ri + 9]
        ri += 9
        if has_down:
            wd_, scd, shd = refs[ri:ri + 3]
            ri += 3
            res = jnp.dot(x, wd_[...], preferred_element_type=jnp.float32)
            res = res * scd[...] + shd[...]
        else:
            res = None
        z = jnp.dot(x, w1[...], preferred_element_type=jnp.float32)
        z = jnp.maximum(z * sc1[...] + sh1[...], 0.0).astype(jnp.bfloat16)
        zp_ref[...] = jnp.zeros(zp_ref.shape, jnp.bfloat16)
        zp_ref[:, 1:h + 1, 8:8 + wd, :] = z.reshape(nb, h, wd, planes)
        taps = []
        for dh in range(3):
            v = zp_ref[:, dh:dh + h, :, :]
            for dw in range(3):
                taps.append(v[:, :, 7 + dw:7 + dw + wd, :].reshape(M, planes))
        acc = jnp.dot(jnp.concatenate(taps, axis=1), w2[...],
                      preferred_element_type=jnp.float32)
        z = jnp.maximum(acc * sc2[...] + sh2[...], 0.0).astype(jnp.bfloat16)
        z = jnp.dot(z, w3[...], preferred_element_type=jnp.float32)
        z = z * sc3[...] + sh3[...]
        if res is None:
            z = z + x.astype(jnp.float32)
        else:
            z = z + res
        x = jnp.maximum(z, 0.0).astype(jnp.bfloat16)
    o_ref[...] = x.reshape(o_ref.shape)


def _fused_layer(x, v, prefixes, planes):
    n, h, wd, cin = x.shape
    nb = 1
    P2 = _rup(wd + 9, 8)
    cout = 4 * planes

    ins = [x]
    in_specs = [pl.BlockSpec((nb, h, wd, cin), lambda b: (b, 0, 0, 0))]
    blocks = []

    def vec(a, npad):
        return jnp.pad(a.reshape(1, -1).astype(jnp.float32),
                       ((0, 0), (0, npad - a.shape[0])))

    def add(arr, spec_shape):
        ins.append(arr)
        in_specs.append(pl.BlockSpec(spec_shape, lambda b: tuple(
            0 for _ in spec_shape)))

    cb = cin
    for p, has_down in prefixes:
        blocks.append(has_down)
        w1 = v[p + 'conv1_w'].reshape(-1, planes)
        if w1.shape[0] != cb:
            w1 = jnp.pad(w1, ((0, cb - w1.shape[0]), (0, 0)))
        add(w1.astype(jnp.bfloat16), (cb, planes))
        add(vec(v[p + 'bn1_scale'], planes), (1, planes))
        add(vec(v[p + 'bn1_shift'], planes), (1, planes))
        w2 = v[p + 'conv2_w'].reshape(9 * planes, planes).astype(jnp.bfloat16)
        add(w2, (9 * planes, planes))
        add(vec(v[p + 'bn2_scale'], planes), (1, planes))
        add(vec(v[p + 'bn2_shift'], planes), (1, planes))
        add(v[p + 'conv3_w'].reshape(planes, cout).astype(jnp.bfloat16),
            (planes, cout))
        add(vec(v[p + 'bn3_scale'], cout), (1, cout))
        add(vec(v[p + 'bn3_shift'], cout), (1, cout))
        if has_down:
            wdn = v[p + 'down_w'].reshape(-1, cout)
            if wdn.shape[0] != cb:
                wdn = jnp.pad(wdn, ((0, cb - wdn.shape[0]), (0, 0)))
            add(wdn.astype(jnp.bfloat16), (cb, cout))
            add(vec(v[p + 'down_bn_scale'], cout), (1, cout))
            add(vec(v[p + 'down_bn_shift'], cout), (1, cout))
        cb = cout

    out = pl.pallas_call(
        functools.partial(_layer_kernel, blocks=blocks, nb=nb, h=h, wd=wd,
                          planes=planes, cin=cin),
        grid=(n // nb,),
        in_specs=in_specs,
        out_specs=pl.BlockSpec((nb, h, wd, cout), lambda b: (b, 0, 0, 0)),
        out_shape=jax.ShapeDtypeStruct((n, h, wd, cout), jnp.bfloat16),
        scratch_shapes=[pltpu.VMEM((nb, h + 2, P2, planes), jnp.bfloat16)],
        compiler_params=pltpu.CompilerParams(
            dimension_semantics=("parallel",),
            vmem_limit_bytes=_VMEM_LIMIT))(*ins)
    return out


def _b0_tail_kernel(zee_ref, zeo_ref, zoe_ref, zoo_ref, xd_ref, w2_ref,
                    sc2_ref, sh2_ref, w3_ref, sc3_ref, sh3_ref, wd_ref,
                    scd_ref, shd_ref, o_ref, *, h2, w2, planes):
    nb = o_ref.shape[0]
    M = nb * h2 * w2
    par = ((zee_ref, zeo_ref), (zoe_ref, zoo_ref))
    taps = []
    for dh in range(3):
        pr, k = dh % 2, dh // 2
        for dw in range(3):
            pc, l = dw % 2, dw // 2
            v = par[pr][pc][:, k:k + h2, :, :]
            taps.append(v[:, :, l:l + w2, :].reshape(M, planes))
    acc = jnp.dot(jnp.concatenate(taps, axis=1), w2_ref[...],
                  preferred_element_type=jnp.float32)
    z = jnp.maximum(acc * sc2_ref[...] + sh2_ref[...], 0.0).astype(jnp.bfloat16)
    z = jnp.dot(z, w3_ref[...], preferred_element_type=jnp.float32)
    z = z * sc3_ref[...] + sh3_ref[...]
    xd = xd_ref[...].reshape(M, xd_ref.shape[-1])
    res = jnp.dot(xd, wd_ref[...], preferred_element_type=jnp.float32)
    res = res * scd_ref[...] + shd_ref[...]
    z = jnp.maximum(z + res, 0.0).astype(jnp.bfloat16)
    o_ref[...] = z.reshape(o_ref.shape)


def _b0_stride2(x, p, v):
    n, h, wd, cin = x.shape
    planes = v[p + 'conv1_w'].shape[3]
    cout = 4 * planes
    h2, wd2 = h // 2, wd // 2
    z1 = _conv_mm(x, v[p + 'conv1_w'], v[p + 'bn1_scale'], v[p + 'bn1_shift'],
                  stride=1, pad=0, relu=True)[..., :planes]
    zp = jnp.pad(z1, ((0, 0), (1, 1), (1, 1), (0, 0)))
    P3 = _rup(wd2 + 1, 8)
    pars = []
    for pr in range(2):
        for pc in range(2):
            a = zp[:, pr::2, pc::2, :]
            a = jnp.pad(a, ((0, 0), (0, h2 + 1 - a.shape[1]),
                            (0, P3 - a.shape[2]), (0, 0)))
            pars.append(a)
    xd = x[:, ::2, ::2, :]

    w2 = v[p + 'conv2_w'].reshape(9 * planes, planes).astype(jnp.bfloat16)
    w3 = v[p + 'conv3_w'].reshape(planes, cout).astype(jnp.bfloat16)
    wdn = v[p + 'down_w'].reshape(-1, cout)
    if wdn.shape[0] != cin:
        wdn = jnp.pad(wdn, ((0, cin - wdn.shape[0]), (0, 0)))
    wdn = wdn.astype(jnp.bfloat16)
    vec = lambda a: a.reshape(1, -1).astype(jnp.float32)

    nb = 1 if h2 * wd2 >= 1024 else 2
    par_spec = pl.BlockSpec((nb, h2 + 1, P3, planes), lambda b: (b, 0, 0, 0))
    cst = lambda r, c: pl.BlockSpec((r, c), lambda b: (0, 0))
    out = pl.pallas_call(
        functools.partial(_b0_tail_kernel, h2=h2, w2=wd2, planes=planes),
        grid=(n // nb,),
        in_specs=[par_spec] * 4 + [
            pl.BlockSpec((nb, h2, wd2, cin), lambda b: (b, 0, 0, 0)),
            cst(9 * planes, planes), cst(1, planes), cst(1, planes),
            cst(planes, cout), cst(1, cout), cst(1, cout),
            cst(cin, cout), cst(1, cout), cst(1, cout),
        ],
        out_specs=pl.BlockSpec((nb, h2, wd2, cout), lambda b: (b, 0, 0, 0)),
        out_shape=jax.ShapeDtypeStruct((n, h2, wd2, cout), jnp.bfloat16),
        compiler_params=pltpu.CompilerParams(
            dimension_semantics=("parallel",),
            vmem_limit_bytes=_VMEM_LIMIT))(
        *pars, xd, w2, vec(v[p + 'bn2_scale']), vec(v[p + 'bn2_shift']),
        w3, vec(v[p + 'bn3_scale']), vec(v[p + 'bn3_shift']),
        wdn, vec(v[p + 'down_bn_scale']), vec(v[p + 'down_bn_shift']))
    return out


def _stem_kernel(pe_ref, po_ref, w_ref, sc_ref, sh_ref, o_ref, *, tm, wo):
    base = pl.program_id(1) * tm
    acc = jnp.zeros(o_ref.shape[1:], jnp.float32)
    for k in range(4):
        acc += jnp.dot(pe_ref[0, pl.ds(base + k * wo, tm), :],
                       w_ref[k * 24:(k + 1) * 24, :],
                       preferred_element_type=jnp.float32)
    for k in range(3):
        acc += jnp.dot(po_ref[0, pl.ds(base + k * wo, tm), :],
                       w_ref[(4 + k) * 24:(5 + k) * 24, :],
                       preferred_element_type=jnp.float32)
    acc = acc * sc_ref[...] + sh_ref[...]
    o_ref[0] = jnp.maximum(acc, 0.0).astype(o_ref.dtype)


def _stem_conv(x_nchw, w, scale, shift):
    n, cin, H, W = x_nchw.shape
    kh, kw, _, cout = w.shape
    Ho, Wo = H // 2, W // 2
    Np = _rup(cout, 128)

    xb = x_nchw.astype(jnp.bfloat16)
    xp = jnp.pad(xb, ((0, 0), (0, 0), (0, 0), (3, 3)))
    Wp = W + 6
    g = np.zeros((Wp, kw * Wo), np.float32)
    for dw in range(kw):
        for j in range(Wo):
            g[2 * j + dw, dw * Wo + j] = 1.0
    p1 = _matmul(xp.reshape(n * cin * H, Wp),
                 jnp.asarray(g, jnp.bfloat16))[:, :kw * Wo]
    p1 = p1.reshape(n, cin, H, kw, Wo).transpose(0, 2, 4, 1, 3)
    p1 = p1.reshape(n, H, Wo, cin * kw)
    p1 = jnp.pad(p1, ((0, 0), (3, 3), (0, 0), (0, 24 - cin * kw)))
    pe = p1[:, 0::2].reshape(n, (H + 6) // 2 * Wo, 24)
    po = p1[:, 1::2].reshape(n, (H + 6) // 2 * Wo, 24)
    Mi = (H + 6) // 2 * Wo
    Mip = _rup(Mi + 3 * Wo + 256, 256)
    pe = jnp.pad(pe, ((0, 0), (0, Mip - Mi), (0, 0)))
    po = jnp.pad(po, ((0, 0), (0, Mip - Mi), (0, 0)))

    taps = []
    for dh in (0, 2, 4, 6, 1, 3, 5):
        wt = jnp.transpose(w[dh], (1, 0, 2)).reshape(cin * kw, cout)
        taps.append(jnp.pad(wt, ((0, 24 - cin * kw), (0, Np - cout))))
    wb = jnp.concatenate(taps, axis=0).astype(jnp.bfloat16)
    sc = jnp.pad(scale.reshape(1, cout).astype(jnp.float32),
                 ((0, 0), (0, Np - cout)))
    sh = jnp.pad(shift.reshape(1, cout).astype(jnp.float32),
                 ((0, 0), (0, Np - cout)))

    tm = 256
    Mo = Ho * Wo
    out = pl.pallas_call(
        functools.partial(_stem_kernel, tm=tm, wo=Wo),
        grid=(n, Mo // tm),
        in_specs=[
            pl.BlockSpec((1, Mip, 24), lambda nn, ii: (nn, 0, 0)),
            pl.BlockSpec((1, Mip, 24), lambda nn, ii: (nn, 0, 0)),
            pl.BlockSpec((7 * 24, Np), lambda nn, ii: (0, 0)),
            pl.BlockSpec((1, Np), lambda nn, ii: (0, 0)),
            pl.BlockSpec((1, Np), lambda nn, ii: (0, 0)),
        ],
        out_specs=pl.BlockSpec((1, tm, Np), lambda nn, ii: (nn, ii, 0)),
        out_shape=jax.ShapeDtypeStruct((n, Mo, Np), jnp.bfloat16),
        compiler_params=pltpu.CompilerParams(
            dimension_semantics=("parallel", "arbitrary"),
            vmem_limit_bytes=_VMEM_LIMIT))(pe, po, wb, sc, sh)
    return out.reshape(n, Ho, Wo, Np)


def _im2col(x, kh, kw, stride, pad):
    n, h, w, c = x.shape
    xp = jnp.pad(x, ((0, 0), (pad, pad), (pad, pad), (0, 0)))
    ho = (h + 2 * pad - kh) // stride + 1
    wo = (w + 2 * pad - kw) // stride + 1
    cols = []
    for dh in range(kh):
        for dw in range(kw):
            cols.append(xp[:, dh:dh + ho * stride:stride,
                           dw:dw + wo * stride:stride, :])
    patches = jnp.concatenate(cols, axis=-1).reshape(n * ho * wo, kh * kw * c)
    return patches, (n, ho, wo)


def _conv_mm(x, w, scale, shift, *, stride, pad, relu, residual=None):
    kh, kw, cin_w, cout = w.shape
    cin_x = x.shape[-1]
    if cin_x != cin_w:
        w = jnp.pad(w, ((0, 0), (0, 0), (0, cin_x - cin_w), (0, 0)))
    if kh == 1 and kw == 1 and pad == 0:
        xs = x[:, ::stride, ::stride, :] if stride > 1 else x
        n, ho, wo, _ = xs.shape
        patches = xs.reshape(n * ho * wo, cin_x)
    else:
        patches, (n, ho, wo) = _im2col(x, kh, kw, stride, pad)
    wm = w.reshape(-1, cout)
    res_flat = None
    if residual is not None:
        res_flat = residual.reshape(n * ho * wo, residual.shape[-1])
    out = _matmul(patches, wm, scale=scale, shift=shift,
                  residual=res_flat, relu=relu)
    return out.reshape(n, ho, wo, out.shape[-1])


def _maxpool_3x3_s2_p1(x):
    init = jnp.array(-jnp.inf, dtype=x.dtype)
    return jax.lax.reduce_window(x, init, jax.lax.max,
                                 window_dimensions=(1, 3, 3, 1),
                                 window_strides=(1, 2, 2, 1),
                                 padding=((0, 0), (1, 1), (1, 1), (0, 0)))


def _bottleneck(x, p, v, stride):
    out = _conv_mm(x, v[p + 'conv1_w'], v[p + 'bn1_scale'], v[p + 'bn1_shift'],
                   stride=1, pad=0, relu=True)
    if stride == 1:
        out = _conv3x3_s1(out, v[p + 'conv2_w'],
                          v[p + 'bn2_scale'], v[p + 'bn2_shift'])
    else:
        out = _conv_mm(out, v[p + 'conv2_w'], v[p + 'bn2_scale'],
                       v[p + 'bn2_shift'], stride=stride, pad=1, relu=True)
    if p + 'down_w' in v:
        identity = _conv_mm(x, v[p + 'down_w'], v[p + 'down_bn_scale'],
                            v[p + 'down_bn_shift'],
                            stride=stride, pad=0, relu=False)
    else:
        identity = x
    out = _conv_mm(out, v[p + 'conv3_w'], v[p + 'bn3_scale'], v[p + 'bn3_shift'],
                   stride=1, pad=0, relu=True, residual=identity)
    return out


def _interp_matrix(out_size, in_size):
    scale = in_size / out_size
    o = np.arange(out_size)
    src = np.maximum((o + 0.5) * scale - 0.5, 0.0)
    i0 = np.minimum(np.floor(src).astype(np.int64), in_size - 1)
    i1 = np.minimum(i0 + 1, in_size - 1)
    w1 = (src - i0).astype(np.float32)
    w0 = (1.0 - w1).astype(np.float32)
    mat = np.zeros((out_size, in_size), np.float32)
    mat[o, i0] += w0
    mat[o, i1] += w1
    return mat


def _head_kernel(w_ref, x_ref, b_ref, o_ref):
    acc = jnp.dot(w_ref[...], x_ref[...], preferred_element_type=jnp.float32)
    o_ref[...] = jnp.maximum(acc + b_ref[...], 0.0).astype(o_ref.dtype)


def _head_classmajor(feat, w, b, n, h, wd):
    cin = feat.shape[-1]
    cin_w = w.shape[0]
    M = n * h * wd
    xt = feat.reshape(M, cin).T
    wt = w.T.astype(jnp.bfloat16)
    if cin_w != cin:
        wt = jnp.pad(wt, ((0, 0), (0, cin - cin_w)))
    wt = jnp.pad(wt, ((0, 1), (0, 0)))
    bc = jnp.pad(b.astype(jnp.float32).reshape(7, 1), ((0, 1), (0, 0)))
    tn = 256 if M % 256 == 0 else 128
    out = pl.pallas_call(
        _head_kernel,
        grid=(M // tn,),
        in_specs=[pl.BlockSpec((8, cin), lambda j: (0, 0)),
                  pl.BlockSpec((cin, tn), lambda j: (0, j)),
                  pl.BlockSpec((8, 1), lambda j: (0, 0))],
        out_specs=pl.BlockSpec((8, tn), lambda j: (0, j)),
        out_shape=jax.ShapeDtypeStruct((8, M), jnp.bfloat16),
        compiler_params=pltpu.CompilerParams(
            dimension_semantics=("parallel",),
            vmem_limit_bytes=_VMEM_LIMIT))(wt.astype(jnp.bfloat16),
                                           xt.astype(jnp.bfloat16), bc)
    return out.reshape(8, n, h, wd)


def _tail_kernel(f0_ref, f1_ref, f2_ref, a0_ref, b0_ref, a1_ref, b1_ref,
                 a2_ref, b2_ref, o_ref):
    acc = jnp.zeros((256, 256), jnp.float32)
    for f_ref, a_ref, b_ref in ((f0_ref, a0_ref, b0_ref),
                                (f1_ref, a1_ref, b1_ref),
                                (f2_ref, a2_ref, b2_ref)):
        t = jnp.dot(f_ref[0, 0], b_ref[...],
                    preferred_element_type=jnp.float32)
        acc += jnp.dot(a_ref[...], t.astype(jnp.bfloat16),
                       preferred_element_type=jnp.float32)
    o_ref[0, 0] = acc


def _fused_tail(relu2, relu3, relu4, fw0, fb0, fw1, fb1, fw2, fb2, n):
    h0 = _head_classmajor(relu2, fw0, fb0, n, 32, 32)
    h1 = _head_classmajor(relu3, fw1, fb1, n, 16, 16)
    h2 = _head_classmajor(relu4, fw2, fb2, n, 8, 8)
    h1 = jnp.pad(h1, ((0, 0), (0, 0), (0, 16), (0, 16)))
    h2 = jnp.pad(h2, ((0, 0), (0, 0), (0, 24), (0, 24)))

    m8 = _interp_matrix(256, 32)
    a1m = m8 @ _interp_matrix(32, 16)
    a2m = m8 @ _interp_matrix(32, 8)
    pad_to32 = lambda m: np.pad(m, ((0, 0), (0, 32 - m.shape[1])))
    a0 = jnp.asarray(m8, jnp.bfloat16)
    a1 = jnp.asarray(pad_to32(a1m), jnp.bfloat16)
    a2 = jnp.asarray(pad_to32(a2m), jnp.bfloat16)
    b0 = jnp.asarray(m8.T, jnp.bfloat16)
    b1 = jnp.asarray(pad_to32(a1m).T, jnp.bfloat16)
    b2 = jnp.asarray(pad_to32(a2m).T, jnp.bfloat16)

    full = lambda r, c: pl.BlockSpec((r, c), lambda nn, cc: (0, 0))
    f_spec = pl.BlockSpec((1, 1, 32, 32), lambda nn, cc: (cc, nn, 0, 0))
    out = pl.pallas_call(
        _tail_kernel,
        grid=(n, 7),
        in_specs=[
            f_spec, f_spec, f_spec,
            full(256, 32), full(32, 256),
            full(256, 32), full(32, 256),
            full(256, 32), full(32, 256),
        ],
        out_specs=pl.BlockSpec((1, 1, 256, 256), lambda nn, cc: (nn, cc, 0, 0)),
        out_shape=jax.ShapeDtypeStruct((n, 7, 256, 256), jnp.float32),
        compiler_params=pltpu.CompilerParams(
            dimension_semantics=("parallel", "parallel"),
            vmem_limit_bytes=_VMEM_LIMIT))(
        h0, h1, h2, a0, b0, a1, b1, a2, b2)
    return out


def kernel(x, conv1_w, bn1_scale, bn1_shift, L1_b0_conv1_w, L1_b0_bn1_scale, L1_b0_bn1_shift, L1_b0_conv2_w, L1_b0_bn2_scale, L1_b0_bn2_shift, L1_b0_conv3_w, L1_b0_bn3_scale, L1_b0_bn3_shift, L1_b0_down_w, L1_b0_down_bn_scale, L1_b0_down_bn_shift, L1_b1_conv1_w, L1_b1_bn1_scale, L1_b1_bn1_shift, L1_b1_conv2_w, L1_b1_bn2_scale, L1_b1_bn2_shift, L1_b1_conv3_w, L1_b1_bn3_scale, L1_b1_bn3_shift, L1_b2_conv1_w, L1_b2_bn1_scale, L1_b2_bn1_shift, L1_b2_conv2_w, L1_b2_bn2_scale, L1_b2_bn2_shift, L1_b2_conv3_w, L1_b2_bn3_scale, L1_b2_bn3_shift, L2_b0_conv1_w, L2_b0_bn1_scale, L2_b0_bn1_shift, L2_b0_conv2_w, L2_b0_bn2_scale, L2_b0_bn2_shift, L2_b0_conv3_w, L2_b0_bn3_scale, L2_b0_bn3_shift, L2_b0_down_w, L2_b0_down_bn_scale, L2_b0_down_bn_shift, L2_b1_conv1_w, L2_b1_bn1_scale, L2_b1_bn1_shift, L2_b1_conv2_w, L2_b1_bn2_scale, L2_b1_bn2_shift, L2_b1_conv3_w, L2_b1_bn3_scale, L2_b1_bn3_shift, L2_b2_conv1_w, L2_b2_bn1_scale, L2_b2_bn1_shift, L2_b2_conv2_w, L2_b2_bn2_scale, L2_b2_bn2_shift, L2_b2_conv3_w, L2_b2_bn3_scale, L2_b2_bn3_shift, L2_b3_conv1_w, L2_b3_bn1_scale, L2_b3_bn1_shift, L2_b3_conv2_w, L2_b3_bn2_scale, L2_b3_bn2_shift, L2_b3_conv3_w, L2_b3_bn3_scale, L2_b3_bn3_shift, L3_b0_conv1_w, L3_b0_bn1_scale, L3_b0_bn1_shift, L3_b0_conv2_w, L3_b0_bn2_scale, L3_b0_bn2_shift, L3_b0_conv3_w, L3_b0_bn3_scale, L3_b0_bn3_shift, L3_b0_down_w, L3_b0_down_bn_scale, L3_b0_down_bn_shift, L3_b1_conv1_w, L3_b1_bn1_scale, L3_b1_bn1_shift, L3_b1_conv2_w, L3_b1_bn2_scale, L3_b1_bn2_shift, L3_b1_conv3_w, L3_b1_bn3_scale, L3_b1_bn3_shift, L3_b2_conv1_w, L3_b2_bn1_scale, L3_b2_bn1_shift, L3_b2_conv2_w, L3_b2_bn2_scale, L3_b2_bn2_shift, L3_b2_conv3_w, L3_b2_bn3_scale, L3_b2_bn3_shift, L3_b3_conv1_w, L3_b3_bn1_scale, L3_b3_bn1_shift, L3_b3_conv2_w, L3_b3_bn2_scale, L3_b3_bn2_shift, L3_b3_conv3_w, L3_b3_bn3_scale, L3_b3_bn3_shift, L3_b4_conv1_w, L3_b4_bn1_scale, L3_b4_bn1_shift, L3_b4_conv2_w, L3_b4_bn2_scale, L3_b4_bn2_shift, L3_b4_conv3_w, L3_b4_bn3_scale, L3_b4_bn3_shift, L3_b5_conv1_w, L3_b5_bn1_scale, L3_b5_bn1_shift, L3_b5_conv2_w, L3_b5_bn2_scale, L3_b5_bn2_shift, L3_b5_conv3_w, L3_b5_bn3_scale, L3_b5_bn3_shift, L4_b0_conv1_w, L4_b0_bn1_scale, L4_b0_bn1_shift, L4_b0_conv2_w, L4_b0_bn2_scale, L4_b0_bn2_shift, L4_b0_conv3_w, L4_b0_bn3_scale, L4_b0_bn3_shift, L4_b0_down_w, L4_b0_down_bn_scale, L4_b0_down_bn_shift, L4_b1_conv1_w, L4_b1_bn1_scale, L4_b1_bn1_shift, L4_b1_conv2_w, L4_b1_bn2_scale, L4_b1_bn2_shift, L4_b1_conv3_w, L4_b1_bn3_scale, L4_b1_bn3_shift, L4_b2_conv1_w, L4_b2_bn1_scale, L4_b2_bn1_shift, L4_b2_conv2_w, L4_b2_bn2_scale, L4_b2_bn2_shift, L4_b2_conv3_w, L4_b2_bn3_scale, L4_b2_bn3_shift, fconv0_w, fconv0_b, fconv1_w, fconv1_b, fconv2_w, fconv2_b):
    v = dict(locals())
    n = x.shape[0]
    xh = _stem_conv(x, conv1_w, bn1_scale, bn1_shift)
    xh = _maxpool_3x3_s2_p1(xh)

    nblocks = {1: 3, 2: 4, 3: 6, 4: 3}
    planes = {1: 64, 2: 128, 3: 256, 4: 512}
    feats = {}
    for L in (1, 2, 3, 4):
        start = 0
        if L > 1:
            xh = _b0_stride2(xh, "L%d_b0_" % L, v)
            start = 1
        prefixes = [("L%d_b%d_" % (L, b), b == 0)
                    for b in range(start, nblocks[L])]
        xh = _fused_layer(xh, v, prefixes, planes[L])
        feats[L] = xh

    return _fused_tail(feats[2], feats[3], feats[4],
                       fconv0_w, fconv0_b, fconv1_w, fconv1_b,
                       fconv2_w, fconv2_b, n)


# speedup vs baseline: 1.0217x; 1.0217x over previous
import functools
import numpy as np
import jax
import jax.numpy as jnp
from jax.experimental import pallas as pl
from jax.experimental.pallas import tpu as pltpu

_VMEM_LIMIT = 64 * 1024 * 1024


def _rup(x, m):
    return (x + m - 1) // m * m


def _mm_kernel(a_ref, b_ref, sc_ref, sh_ref, res_ref, o_ref, acc_ref, *,
               relu, affine, residual, nk):
    if nk > 1:
        @pl.when(pl.program_id(2) == 0)
        def _():
            acc_ref[...] = jnp.zeros_like(acc_ref)
        acc_ref[...] += jnp.dot(a_ref[...], b_ref[...],
                                preferred_element_type=jnp.float32)

        @pl.when(pl.program_id(2) == nk - 1)
        def _():
            acc = acc_ref[...]
            if affine:
                acc = acc * sc_ref[...] + sh_ref[...]
            if residual:
                acc = acc + res_ref[...].astype(jnp.float32)
            if relu:
                acc = jnp.maximum(acc, 0.0)
            o_ref[...] = acc.astype(o_ref.dtype)
    else:
        acc = jnp.dot(a_ref[...], b_ref[...],
                      preferred_element_type=jnp.float32)
        if affine:
            acc = acc * sc_ref[...] + sh_ref[...]
        if residual:
            acc = acc + res_ref[...].astype(jnp.float32)
        if relu:
            acc = jnp.maximum(acc, 0.0)
        o_ref[...] = acc.astype(o_ref.dtype)


def _matmul(a, b, *, scale=None, shift=None, residual=None, relu=False,
            out_dtype=jnp.bfloat16):
    M, K = a.shape
    Kb, N = b.shape
    assert K == Kb, (a.shape, b.shape)
    Np = _rup(N, 128)
    Mp = _rup(M, 8)

    if Mp >= 512:
        tm = 256
    elif Mp >= 128:
        tm = Mp // 2 if Mp % 2 == 0 and (Mp // 2) % 8 == 0 else Mp
    else:
        tm = Mp
    Mp = _rup(Mp, tm)
    if Np >= 1024:
        tn = 512
    elif Np >= 256:
        tn = 256 if (Mp // tm) * (Np // 256) >= 2 else 128
    else:
        tn = 128
    while Np % tn:
        tn //= 2
    Kp = _rup(K, 8)
    if Kp > 1024:
        tk = next((t for t in (1024, 512, 256) if Kp % t == 0), Kp)
    else:
        tk = Kp
    nk = Kp // tk

    a = a.astype(jnp.bfloat16)
    b = b.astype(jnp.bfloat16)
    if Kp > K:
        a = jnp.pad(a, ((0, 0), (0, Kp - K)))
        b = jnp.pad(b, ((0, Kp - K), (0, 0)))
    if Mp > M:
        a = jnp.pad(a, ((0, Mp - M), (0, 0)))
    if Np > N:
        b = jnp.pad(b, ((0, 0), (0, Np - N)))

    grid = (Mp // tm, Np // tn, nk)
    a_spec = pl.BlockSpec((tm, tk), lambda i, j, k: (i, k))
    b_spec = pl.BlockSpec((tk, tn), lambda i, j, k: (k, j))
    o_spec = pl.BlockSpec((tm, tn), lambda i, j, k: (i, j))
    v_spec = pl.BlockSpec((1, tn), lambda i, j, k: (0, j))

    affine = scale is not None
    has_res = residual is not None
    ins = [a, b]
    in_specs = [a_spec, b_spec]
    if affine:
        sc = jnp.pad(scale.reshape(1, N).astype(jnp.float32),
                     ((0, 0), (0, Np - N)))
        sh = jnp.pad(shift.reshape(1, N).astype(jnp.float32),
                     ((0, 0), (0, Np - N)))
        ins += [sc, sh]
        in_specs += [v_spec, v_spec]
    if has_res:
        res = residual.astype(jnp.bfloat16)
        if res.shape != (Mp, Np):
            res = jnp.pad(res, ((0, Mp - res.shape[0]),
                                (0, Np - res.shape[1])))
        ins.append(res)
        in_specs.append(o_spec)

    kern = functools.partial(_mm_kernel, relu=relu, affine=affine,
                             residual=has_res, nk=nk)
    if not affine:
        kern = lambda a_r, b_r, o_r, ac_r: functools.partial(
            _mm_kernel, relu=relu, affine=False, residual=False, nk=nk)(
                a_r, b_r, None, None, None, o_r, ac_r)
    elif not has_res:
        kern = lambda a_r, b_r, s_r, h_r, o_r, ac_r: functools.partial(
            _mm_kernel, relu=relu, affine=True, residual=False, nk=nk)(
                a_r, b_r, s_r, h_r, None, o_r, ac_r)

    out = pl.pallas_call(
        kern, grid=grid,
        in_specs=in_specs, out_specs=o_spec,
        out_shape=jax.ShapeDtypeStruct((Mp, Np), out_dtype),
        scratch_shapes=[pltpu.VMEM((tm, tn), jnp.float32)],
        compiler_params=pltpu.CompilerParams(
            dimension_semantics=("parallel", "parallel", "arbitrary"),
            vmem_limit_bytes=_VMEM_LIMIT))(*ins)
    if Mp > M:
        out = out[:M]
    return out


def _layer_kernel(*refs, blocks, nb, h, wd, planes, cin):
    x_ref = refs[0]
    o_ref = refs[-2]
    zp_ref = refs[-1]
    M = nb * h * wd
    x = x_ref[...].reshape(M, cin)
    ri = 1
    for has_down in blocks:
        w1, sc1, sh1, w2, sc2, sh2, w3, sc3, sh3 = refs[ri:ri + 9]
        ri += 9
        if has_down:
            wd_, scd, shd = refs[ri:ri + 3]
            ri += 3
            res = jnp.dot(x, wd_[...], preferred_element_type=jnp.float32)
            res = res * scd[...] + shd[...]
        else:
            res = None
        z = jnp.dot(x, w1[...], preferred_element_type=jnp.float32)
        z = jnp.maximum(z * sc1[...] + sh1[...], 0.0).astype(jnp.bfloat16)
        zp_ref[...] = jnp.zeros(zp_ref.shape, jnp.bfloat16)
        zp_ref[:, 1:h + 1, 8:8 + wd, :] = z.reshape(nb, h, wd, planes)
        taps = []
        for dh in range(3):
            v = zp_ref[:, dh:dh + h, :, :]
            for dw in range(3):
                taps.append(v[:, :, 7 + dw:7 + dw + wd, :].reshape(M, planes))
        acc = jnp.dot(jnp.concatenate(taps, axis=1), w2[...],
                      preferred_element_type=jnp.float32)
        z = jnp.maximum(acc * sc2[...] + sh2[...], 0.0).astype(jnp.bfloat16)
        z = jnp.dot(z, w3[...], preferred_element_type=jnp.float32)
        z = z * sc3[...] + sh3[...]
        if res is None:
            z = z + x.astype(jnp.float32)
        else:
            z = z + res
        x = jnp.maximum(z, 0.0).astype(jnp.bfloat16)
    o_ref[...] = x.reshape(o_ref.shape)


def _fused_layer(x, v, prefixes, planes):
    n, h, wd, cin = x.shape
    nb = 1 if h >= 32 else 2
    P2 = _rup(wd + 9, 8)
    cout = 4 * planes

    ins = [x]
    in_specs = [pl.BlockSpec((nb, h, wd, cin), lambda b: (b, 0, 0, 0))]
    blocks = []

    def vec(a, npad):
        return jnp.pad(a.reshape(1, -1).astype(jnp.float32),
                       ((0, 0), (0, npad - a.shape[0])))

    def add(arr, spec_shape):
        ins.append(arr)
        in_specs.append(pl.BlockSpec(spec_shape, lambda b: tuple(
            0 for _ in spec_shape)))

    cb = cin
    for p, has_down in prefixes:
        blocks.append(has_down)
        w1 = v[p + 'conv1_w'].reshape(-1, planes)
        if w1.shape[0] != cb:
            w1 = jnp.pad(w1, ((0, cb - w1.shape[0]), (0, 0)))
        add(w1.astype(jnp.bfloat16), (cb, planes))
        add(vec(v[p + 'bn1_scale'], planes), (1, planes))
        add(vec(v[p + 'bn1_shift'], planes), (1, planes))
        w2 = v[p + 'conv2_w'].reshape(9 * planes, planes).astype(jnp.bfloat16)
        add(w2, (9 * planes, planes))
        add(vec(v[p + 'bn2_scale'], planes), (1, planes))
        add(vec(v[p + 'bn2_shift'], planes), (1, planes))
        add(v[p + 'conv3_w'].reshape(planes, cout).astype(jnp.bfloat16),
            (planes, cout))
        add(vec(v[p + 'bn3_scale'], cout), (1, cout))
        add(vec(v[p + 'bn3_shift'], cout), (1, cout))
        if has_down:
            wdn = v[p + 'down_w'].reshape(-1, cout)
            if wdn.shape[0] != cb:
                wdn = jnp.pad(wdn, ((0, cb - wdn.shape[0]), (0, 0)))
            add(wdn.astype(jnp.bfloat16), (cb, cout))
            add(vec(v[p + 'down_bn_scale'], cout), (1, cout))
            add(vec(v[p + 'down_bn_shift'], cout), (1, cout))
        cb = cout

    out = pl.pallas_call(
        functools.partial(_layer_kernel, blocks=blocks, nb=nb, h=h, wd=wd,
                          planes=planes, cin=cin),
        grid=(n // nb,),
        in_specs=in_specs,
        out_specs=pl.BlockSpec((nb, h, wd, cout), lambda b: (b, 0, 0, 0)),
        out_shape=jax.ShapeDtypeStruct((n, h, wd, cout), jnp.bfloat16),
        scratch_shapes=[pltpu.VMEM((nb, h + 2, P2, planes), jnp.bfloat16)],
        compiler_params=pltpu.CompilerParams(
            dimension_semantics=("parallel",),
            vmem_limit_bytes=_VMEM_LIMIT))(*ins)
    return out


def _b0_tail_kernel(zee_ref, zeo_ref, zoe_ref, zoo_ref, xd_ref, w2_ref,
                    sc2_ref, sh2_ref, w3_ref, sc3_ref, sh3_ref, wd_ref,
                    scd_ref, shd_ref, o_ref, *, h2, w2, planes):
    nb = o_ref.shape[0]
    M = nb * h2 * w2
    par = ((zee_ref, zeo_ref), (zoe_ref, zoo_ref))
    taps = []
    for dh in range(3):
        pr, k = dh % 2, dh // 2
        for dw in range(3):
            pc, l = dw % 2, dw // 2
            v = par[pr][pc][:, k:k + h2, :, :]
            taps.append(v[:, :, l:l + w2, :].reshape(M, planes))
    acc = jnp.dot(jnp.concatenate(taps, axis=1), w2_ref[...],
                  preferred_element_type=jnp.float32)
    z = jnp.maximum(acc * sc2_ref[...] + sh2_ref[...], 0.0).astype(jnp.bfloat16)
    z = jnp.dot(z, w3_ref[...], preferred_element_type=jnp.float32)
    z = z * sc3_ref[...] + sh3_ref[...]
    xd = xd_ref[...].reshape(M, xd_ref.shape[-1])
    res = jnp.dot(xd, wd_ref[...], preferred_element_type=jnp.float32)
    res = res * scd_ref[...] + shd_ref[...]
    z = jnp.maximum(z + res, 0.0).astype(jnp.bfloat16)
    o_ref[...] = z.reshape(o_ref.shape)


def _b0_stride2(x, p, v):
    n, h, wd, cin = x.shape
    planes = v[p + 'conv1_w'].shape[3]
    cout = 4 * planes
    h2, wd2 = h // 2, wd // 2
    z1 = _conv_mm(x, v[p + 'conv1_w'], v[p + 'bn1_scale'], v[p + 'bn1_shift'],
                  stride=1, pad=0, relu=True)[..., :planes]
    zp = jnp.pad(z1, ((0, 0), (1, 1), (1, 1), (0, 0)))
    P3 = _rup(wd2 + 1, 8)
    pars = []
    for pr in range(2):
        for pc in range(2):
            a = zp[:, pr::2, pc::2, :]
            a = jnp.pad(a, ((0, 0), (0, h2 + 1 - a.shape[1]),
                            (0, P3 - a.shape[2]), (0, 0)))
            pars.append(a)
    xd = x[:, ::2, ::2, :]

    w2 = v[p + 'conv2_w'].reshape(9 * planes, planes).astype(jnp.bfloat16)
    w3 = v[p + 'conv3_w'].reshape(planes, cout).astype(jnp.bfloat16)
    wdn = v[p + 'down_w'].reshape(-1, cout)
    if wdn.shape[0] != cin:
        wdn = jnp.pad(wdn, ((0, cin - wdn.shape[0]), (0, 0)))
    wdn = wdn.astype(jnp.bfloat16)
    vec = lambda a: a.reshape(1, -1).astype(jnp.float32)

    nb = 1 if h2 * wd2 >= 1024 else 2
    par_spec = pl.BlockSpec((nb, h2 + 1, P3, planes), lambda b: (b, 0, 0, 0))
    cst = lambda r, c: pl.BlockSpec((r, c), lambda b: (0, 0))
    out = pl.pallas_call(
        functools.partial(_b0_tail_kernel, h2=h2, w2=wd2, planes=planes),
        grid=(n // nb,),
        in_specs=[par_spec] * 4 + [
            pl.BlockSpec((nb, h2, wd2, cin), lambda b: (b, 0, 0, 0)),
            cst(9 * planes, planes), cst(1, planes), cst(1, planes),
            cst(planes, cout), cst(1, cout), cst(1, cout),
            cst(cin, cout), cst(1, cout), cst(1, cout),
        ],
        out_specs=pl.BlockSpec((nb, h2, wd2, cout), lambda b: (b, 0, 0, 0)),
        out_shape=jax.ShapeDtypeStruct((n, h2, wd2, cout), jnp.bfloat16),
        compiler_params=pltpu.CompilerParams(
            dimension_semantics=("parallel",),
            vmem_limit_bytes=_VMEM_LIMIT))(
        *pars, xd, w2, vec(v[p + 'bn2_scale']), vec(v[p + 'bn2_shift']),
        w3, vec(v[p + 'bn3_scale']), vec(v[p + 'bn3_shift']),
        wdn, vec(v[p + 'down_bn_scale']), vec(v[p + 'down_bn_shift']))
    return out


def _stem_kernel(pe_ref, po_ref, w_ref, sc_ref, sh_ref, o_ref, *, tm, wo):
    base = pl.program_id(1) * tm
    acc = jnp.zeros(o_ref.shape[1:], jnp.float32)
    for k in range(4):
        acc += jnp.dot(pe_ref[0, pl.ds(base + k * wo, tm), :],
                       w_ref[k * 24:(k + 1) * 24, :],
                       preferred_element_type=jnp.float32)
    for k in range(3):
        acc += jnp.dot(po_ref[0, pl.ds(base + k * wo, tm), :],
                       w_ref[(4 + k) * 24:(5 + k) * 24, :],
                       preferred_element_type=jnp.float32)
    acc = acc * sc_ref[...] + sh_ref[...]
    o_ref[0] = jnp.maximum(acc, 0.0).astype(o_ref.dtype)


def _stem_conv(x_nchw, w, scale, shift):
    n, cin, H, W = x_nchw.shape
    kh, kw, _, cout = w.shape
    Ho, Wo = H // 2, W // 2
    Np = _rup(cout, 128)

    xb = x_nchw.astype(jnp.bfloat16)
    xp = jnp.pad(xb, ((0, 0), (0, 0), (0, 0), (3, 3)))
    Wp = W + 6
    g = np.zeros((Wp, kw * Wo), np.float32)
    for dw in range(kw):
        for j in range(Wo):
            g[2 * j + dw, dw * Wo + j] = 1.0
    p1 = _matmul(xp.reshape(n * cin * H, Wp),
                 jnp.asarray(g, jnp.bfloat16))[:, :kw * Wo]
    p1 = p1.reshape(n, cin, H, kw, Wo).transpose(0, 2, 4, 1, 3)
    p1 = p1.reshape(n, H, Wo, cin * kw)
    p1 = jnp.pad(p1, ((0, 0), (3, 3), (0, 0), (0, 24 - cin * kw)))
    pe = p1[:, 0::2].reshape(n, (H + 6) // 2 * Wo, 24)
    po = p1[:, 1::2].reshape(n, (H + 6) // 2 * Wo, 24)
    Mi = (H + 6) // 2 * Wo
    Mip = _rup(Mi + 3 * Wo + 256, 256)
    pe = jnp.pad(pe, ((0, 0), (0, Mip - Mi), (0, 0)))
    po = jnp.pad(po, ((0, 0), (0, Mip - Mi), (0, 0)))

    taps = []
    for dh in (0, 2, 4, 6, 1, 3, 5):
        wt = jnp.transpose(w[dh], (1, 0, 2)).reshape(cin * kw, cout)
        taps.append(jnp.pad(wt, ((0, 24 - cin * kw), (0, Np - cout))))
    wb = jnp.concatenate(taps, axis=0).astype(jnp.bfloat16)
    sc = jnp.pad(scale.reshape(1, cout).astype(jnp.float32),
                 ((0, 0), (0, Np - cout)))
    sh = jnp.pad(shift.reshape(1, cout).astype(jnp.float32),
                 ((0, 0), (0, Np - cout)))

    tm = 256
    Mo = Ho * Wo
    out = pl.pallas_call(
        functools.partial(_stem_kernel, tm=tm, wo=Wo),
        grid=(n, Mo // tm),
        in_specs=[
            pl.BlockSpec((1, Mip, 24), lambda nn, ii: (nn, 0, 0)),
            pl.BlockSpec((1, Mip, 24), lambda nn, ii: (nn, 0, 0)),
            pl.BlockSpec((7 * 24, Np), lambda nn, ii: (0, 0)),
            pl.BlockSpec((1, Np), lambda nn, ii: (0, 0)),
            pl.BlockSpec((1, Np), lambda nn, ii: (0, 0)),
        ],
        out_specs=pl.BlockSpec((1, tm, Np), lambda nn, ii: (nn, ii, 0)),
        out_shape=jax.ShapeDtypeStruct((n, Mo, Np), jnp.bfloat16),
        compiler_params=pltpu.CompilerParams(
            dimension_semantics=("parallel", "arbitrary"),
            vmem_limit_bytes=_VMEM_LIMIT))(pe, po, wb, sc, sh)
    return out.reshape(n, Ho, Wo, Np)


def _conv_mm(x, w, scale, shift, *, stride, pad, relu, residual=None):
    kh, kw, cin_w, cout = w.shape
    assert kh == 1 and kw == 1 and pad == 0 and stride == 1
    cin_x = x.shape[-1]
    if cin_x != cin_w:
        w = jnp.pad(w, ((0, 0), (0, 0), (0, cin_x - cin_w), (0, 0)))
    n, ho, wo, _ = x.shape
    patches = x.reshape(n * ho * wo, cin_x)
    out = _matmul(patches, w.reshape(-1, cout), scale=scale, shift=shift,
                  relu=relu)
    return out.reshape(n, ho, wo, out.shape[-1])


def _maxpool_3x3_s2_p1(x):
    init = jnp.array(-jnp.inf, dtype=x.dtype)
    return jax.lax.reduce_window(x, init, jax.lax.max,
                                 window_dimensions=(1, 3, 3, 1),
                                 window_strides=(1, 2, 2, 1),
                                 padding=((0, 0), (1, 1), (1, 1), (0, 0)))


def _interp_matrix(out_size, in_size):
    scale = in_size / out_size
    o = np.arange(out_size)
    src = np.maximum((o + 0.5) * scale - 0.5, 0.0)
    i0 = np.minimum(np.floor(src).astype(np.int64), in_size - 1)
    i1 = np.minimum(i0 + 1, in_size - 1)
    w1 = (src - i0).astype(np.float32)
    w0 = (1.0 - w1).astype(np.float32)
    mat = np.zeros((out_size, in_size), np.float32)
    mat[o, i0] += w0
    mat[o, i1] += w1
    return mat


def _head_kernel(w_ref, x_ref, b_ref, o_ref):
    acc = jnp.dot(w_ref[...], x_ref[...], preferred_element_type=jnp.float32)
    o_ref[...] = jnp.maximum(acc + b_ref[...], 0.0).astype(o_ref.dtype)


def _head_classmajor(feat, w, b, n, h, wd):
    cin = feat.shape[-1]
    cin_w = w.shape[0]
    M = n * h * wd
    xt = feat.reshape(M, cin).T
    wt = w.T.astype(jnp.bfloat16)
    if cin_w != cin:
        wt = jnp.pad(wt, ((0, 0), (0, cin - cin_w)))
    wt = jnp.pad(wt, ((0, 1), (0, 0)))
    bc = jnp.pad(b.astype(jnp.float32).reshape(7, 1), ((0, 1), (0, 0)))
    tn = 256 if M % 256 == 0 else 128
    out = pl.pallas_call(
        _head_kernel,
        grid=(M // tn,),
        in_specs=[pl.BlockSpec((8, cin), lambda j: (0, 0)),
                  pl.BlockSpec((cin, tn), lambda j: (0, j)),
                  pl.BlockSpec((8, 1), lambda j: (0, 0))],
        out_specs=pl.BlockSpec((8, tn), lambda j: (0, j)),
        out_shape=jax.ShapeDtypeStruct((8, M), jnp.bfloat16),
        compiler_params=pltpu.CompilerParams(
            dimension_semantics=("parallel",),
            vmem_limit_bytes=_VMEM_LIMIT))(wt.astype(jnp.bfloat16),
                                           xt.astype(jnp.bfloat16), bc)
    return out.reshape(8, n, h, wd)


def _tail_kernel(f0_ref, f1_ref, f2_ref, a0_ref, b0_ref, a1_ref, b1_ref,
                 a2_ref, b2_ref, o_ref):
    acc = jnp.zeros((256, 256), jnp.float32)
    for f_ref, a_ref, b_ref in ((f0_ref, a0_ref, b0_ref),
                                (f1_ref, a1_ref, b1_ref),
                                (f2_ref, a2_ref, b2_ref)):
        t = jnp.dot(f_ref[0, 0], b_ref[...],
                    preferred_element_type=jnp.float32)
        acc += jnp.dot(a_ref[...], t.astype(jnp.bfloat16),
                       preferred_element_type=jnp.float32)
    o_ref[0, 0] = acc


def _fused_tail(relu2, relu3, relu4, fw0, fb0, fw1, fb1, fw2, fb2, n):
    h0 = _head_classmajor(relu2, fw0, fb0, n, 32, 32)
    h1 = _head_classmajor(relu3, fw1, fb1, n, 16, 16)
    h2 = _head_classmajor(relu4, fw2, fb2, n, 8, 8)
    h1 = jnp.pad(h1, ((0, 0), (0, 0), (0, 16), (0, 16)))
    h2 = jnp.pad(h2, ((0, 0), (0, 0), (0, 24), (0, 24)))

    m8 = _interp_matrix(256, 32)
    a1m = m8 @ _interp_matrix(32, 16)
    a2m = m8 @ _interp_matrix(32, 8)
    pad_to32 = lambda m: np.pad(m, ((0, 0), (0, 32 - m.shape[1])))
    a0 = jnp.asarray(m8, jnp.bfloat16)
    a1 = jnp.asarray(pad_to32(a1m), jnp.bfloat16)
    a2 = jnp.asarray(pad_to32(a2m), jnp.bfloat16)
    b0 = jnp.asarray(m8.T, jnp.bfloat16)
    b1 = jnp.asarray(pad_to32(a1m).T, jnp.bfloat16)
    b2 = jnp.asarray(pad_to32(a2m).T, jnp.bfloat16)

    full = lambda r, c: pl.BlockSpec((r, c), lambda nn, cc: (0, 0))
    f_spec = pl.BlockSpec((1, 1, 32, 32), lambda nn, cc: (cc, nn, 0, 0))
    out = pl.pallas_call(
        _tail_kernel,
        grid=(n, 7),
        in_specs=[
            f_spec, f_spec, f_spec,
            full(256, 32), full(32, 256),
            full(256, 32), full(32, 256),
            full(256, 32), full(32, 256),
        ],
        out_specs=pl.BlockSpec((1, 1, 256, 256), lambda nn, cc: (nn, cc, 0, 0)),
        out_shape=jax.ShapeDtypeStruct((n, 7, 256, 256), jnp.float32),
        compiler_params=pltpu.CompilerParams(
            dimension_semantics=("parallel", "parallel"),
            vmem_limit_bytes=_VMEM_LIMIT))(
        h0, h1, h2, a0, b0, a1, b1, a2, b2)
    return out


def kernel(x, conv1_w, bn1_scale, bn1_shift, L1_b0_conv1_w, L1_b0_bn1_scale, L1_b0_bn1_shift, L1_b0_conv2_w, L1_b0_bn2_scale, L1_b0_bn2_shift, L1_b0_conv3_w, L1_b0_bn3_scale, L1_b0_bn3_shift, L1_b0_down_w, L1_b0_down_bn_scale, L1_b0_down_bn_shift, L1_b1_conv1_w, L1_b1_bn1_scale, L1_b1_bn1_shift, L1_b1_conv2_w, L1_b1_bn2_scale, L1_b1_bn2_shift, L1_b1_conv3_w, L1_b1_bn3_scale, L1_b1_bn3_shift, L1_b2_conv1_w, L1_b2_bn1_scale, L1_b2_bn1_shift, L1_b2_conv2_w, L1_b2_bn2_scale, L1_b2_bn2_shift, L1_b2_conv3_w, L1_b2_bn3_scale, L1_b2_bn3_shift, L2_b0_conv1_w, L2_b0_bn1_scale, L2_b0_bn1_shift, L2_b0_conv2_w, L2_b0_bn2_scale, L2_b0_bn2_shift, L2_b0_conv3_w, L2_b0_bn3_scale, L2_b0_bn3_shift, L2_b0_down_w, L2_b0_down_bn_scale, L2_b0_down_bn_shift, L2_b1_conv1_w, L2_b1_bn1_scale, L2_b1_bn1_shift, L2_b1_conv2_w, L2_b1_bn2_scale, L2_b1_bn2_shift, L2_b1_conv3_w, L2_b1_bn3_scale, L2_b1_bn3_shift, L2_b2_conv1_w, L2_b2_bn1_scale, L2_b2_bn1_shift, L2_b2_conv2_w, L2_b2_bn2_scale, L2_b2_bn2_shift, L2_b2_conv3_w, L2_b2_bn3_scale, L2_b2_bn3_shift, L2_b3_conv1_w, L2_b3_bn1_scale, L2_b3_bn1_shift, L2_b3_conv2_w, L2_b3_bn2_scale, L2_b3_bn2_shift, L2_b3_conv3_w, L2_b3_bn3_scale, L2_b3_bn3_shift, L3_b0_conv1_w, L3_b0_bn1_scale, L3_b0_bn1_shift, L3_b0_conv2_w, L3_b0_bn2_scale, L3_b0_bn2_shift, L3_b0_conv3_w, L3_b0_bn3_scale, L3_b0_bn3_shift, L3_b0_down_w, L3_b0_down_bn_scale, L3_b0_down_bn_shift, L3_b1_conv1_w, L3_b1_bn1_scale, L3_b1_bn1_shift, L3_b1_conv2_w, L3_b1_bn2_scale, L3_b1_bn2_shift, L3_b1_conv3_w, L3_b1_bn3_scale, L3_b1_bn3_shift, L3_b2_conv1_w, L3_b2_bn1_scale, L3_b2_bn1_shift, L3_b2_conv2_w, L3_b2_bn2_scale, L3_b2_bn2_shift, L3_b2_conv3_w, L3_b2_bn3_scale, L3_b2_bn3_shift, L3_b3_conv1_w, L3_b3_bn1_scale, L3_b3_bn1_shift, L3_b3_conv2_w, L3_b3_bn2_scale, L3_b3_bn2_shift, L3_b3_conv3_w, L3_b3_bn3_scale, L3_b3_bn3_shift, L3_b4_conv1_w, L3_b4_bn1_scale, L3_b4_bn1_shift, L3_b4_conv2_w, L3_b4_bn2_scale, L3_b4_bn2_shift, L3_b4_conv3_w, L3_b4_bn3_scale, L3_b4_bn3_shift, L3_b5_conv1_w, L3_b5_bn1_scale, L3_b5_bn1_shift, L3_b5_conv2_w, L3_b5_bn2_scale, L3_b5_bn2_shift, L3_b5_conv3_w, L3_b5_bn3_scale, L3_b5_bn3_shift, L4_b0_conv1_w, L4_b0_bn1_scale, L4_b0_bn1_shift, L4_b0_conv2_w, L4_b0_bn2_scale, L4_b0_bn2_shift, L4_b0_conv3_w, L4_b0_bn3_scale, L4_b0_bn3_shift, L4_b0_down_w, L4_b0_down_bn_scale, L4_b0_down_bn_shift, L4_b1_conv1_w, L4_b1_bn1_scale, L4_b1_bn1_shift, L4_b1_conv2_w, L4_b1_bn2_scale, L4_b1_bn2_shift, L4_b1_conv3_w, L4_b1_bn3_scale, L4_b1_bn3_shift, L4_b2_conv1_w, L4_b2_bn1_scale, L4_b2_bn1_shift, L4_b2_conv2_w, L4_b2_bn2_scale, L4_b2_bn2_shift, L4_b2_conv3_w, L4_b2_bn3_scale, L4_b2_bn3_shift, fconv0_w, fconv0_b, fconv1_w, fconv1_b, fconv2_w, fconv2_b):
    v = dict(locals())
    n = x.shape[0]
    xh = _stem_conv(x, conv1_w, bn1_scale, bn1_shift)
    xh = _maxpool_3x3_s2_p1(xh)

    nblocks = {1: 3, 2: 4, 3: 6, 4: 3}
    planes = {1: 64, 2: 128, 3: 256, 4: 512}
    feats = {}
    for L in (1, 2, 3, 4):
        start = 0
        if L > 1:
            xh = _b0_stride2(xh, "L%d_b0_" % L, v)
            start = 1
        prefixes = [("L%d_b%d_" % (L, b), b == 0)
                    for b in range(start, nblocks[L])]
        xh = _fused_layer(xh, v, prefixes, planes[L])
        feats[L] = xh

    return _fused_tail(feats[2], feats[3], feats[4],
                       fconv0_w, fconv0_b, fconv1_w, fconv1_b,
                       fconv2_w, fconv2_b, n)
